# baseline (speedup 1.0000x reference)
import functools

import jax
import jax.numpy as jnp
from jax import lax
from jax.experimental import pallas as pl
from jax.experimental.pallas import tpu as pltpu
from jax.experimental.pallas import tpu_sc as plsc

BATCH = 16384
FIELDS = 26
EMBED_DIM = 32
NUM_NODES = 1000000

NUM_CORES = 2
NUM_SUBCORES = 16
NUM_WORKERS = NUM_CORES * NUM_SUBCORES

TOTAL = BATCH * FIELDS
PER_WORKER = TOTAL // NUM_WORKERS
CHUNK = 128
GROUP = 13
ROWS_PER_GROUP = CHUNK * GROUP
NUM_GROUPS = PER_WORKER // ROWS_PER_GROUP
NUM_CHUNKS = PER_WORKER // CHUNK

RCOLS = 4096
RGRID = 62
V_QUARTER = RCOLS * RGRID
NODES_LIN = 4 * V_QUARTER

assert PER_WORKER * NUM_WORKERS == TOTAL
assert ROWS_PER_GROUP * NUM_GROUPS == PER_WORKER


def _repack_block(w0, w1, w2, w3, out_ref):
    out_ref[:, 0:32] = jnp.transpose(w0[...])
    out_ref[:, 32:64] = jnp.transpose(w1[...])
    out_ref[:, 64:96] = jnp.transpose(w2[...])
    out_ref[:, 96:128] = jnp.transpose(w3[...])


def _repack(wt):
    max_blk = NUM_NODES // RCOLS
    specs = [
        pl.BlockSpec(
            (EMBED_DIM, RCOLS),
            functools.partial(
                lambda q, i: (0, jnp.minimum(q * RGRID + i, max_blk)), q
            ),
        )
        for q in range(4)
    ]
    return pl.pallas_call(
        _repack_block,
        grid=(RGRID,),
        in_specs=specs,
        out_specs=pl.BlockSpec((RCOLS, 128), lambda i: (i, 0)),
        out_shape=jax.ShapeDtypeStruct((V_QUARTER, 128), jnp.float32),
    )(wt, wt, wt, wt)


B_PER_WORKER = BATCH // NUM_WORKERS
B_PER_GROUP = B_PER_WORKER // NUM_GROUPS


@functools.partial(
    pl.kernel,
    mesh=plsc.VectorSubcoreMesh(core_axis_name="c", subcore_axis_name="s"),
    out_type=jax.ShapeDtypeStruct((FIELDS, EMBED_DIM, BATCH), jnp.float32),
    scratch_types=[
        pltpu.VMEM((NUM_CHUNKS, CHUNK), jnp.int32),
        pltpu.VMEM((ROWS_PER_GROUP, EMBED_DIM), jnp.float32),
        pltpu.VMEM((2, EMBED_DIM, B_PER_GROUP), jnp.float32),
        pltpu.SemaphoreType.DMA,
        pltpu.SemaphoreType.DMA,
        pltpu.SemaphoreType.DMA,
    ],
    compiler_params=pltpu.CompilerParams(
        use_tc_tiling_on_sc=False, needs_layout_passes=False
    ),
)
def _gather_kernel(idx_hbm, table_hbm, out_hbm, idx_v, rows_v, tbuf, gsem, osem0, osem1):
    wid = lax.axis_index("s") * NUM_CORES + lax.axis_index("c")
    b_base = wid * B_PER_WORKER
    pltpu.sync_copy(idx_hbm.at[wid], idx_v)
    iota26 = lax.iota(jnp.int32, 16) * FIELDS
    osems = (osem0, osem1)

    def transpose_field(f, par):
        for bc in range(B_PER_GROUP // 16):
            row_idx = iota26 + (bc * 16 * FIELDS + f)
            for d in range(EMBED_DIM):
                vals = plsc.load_gather(
                    rows_v, [row_idx, jnp.full((16,), d, jnp.int32)]
                )
                tbuf[par, d, pl.ds(bc * 16, 16)] = vals

    def group_body(g, _):
        copies = []
        for j in range(GROUP):
            c = pltpu.async_copy(
                table_hbm.at[idx_v.at[g * GROUP + j]],
                rows_v.at[pl.ds(j * CHUNK, CHUNK)],
                gsem,
            )
            copies.append(c)
        for c in copies:
            c.wait()

        b_lo = b_base + g * B_PER_GROUP

        def pair_body(i, _):
            for par in range(2):
                f = 2 * i + par

                @pl.when(i > 0)
                def _wait():
                    pltpu.make_async_copy(
                        tbuf.at[par],
                        out_hbm.at[f, :, pl.ds(b_lo, B_PER_GROUP)],
                        osems[par],
                    ).wait()

                transpose_field(f, par)
                pltpu.async_copy(
                    tbuf.at[par],
                    out_hbm.at[f, :, pl.ds(b_lo, B_PER_GROUP)],
                    osems[par],
                )
            return ()

        lax.fori_loop(0, FIELDS // 2, pair_body, ())
        for par in range(2):
            pltpu.make_async_copy(
                tbuf.at[par],
                out_hbm.at[FIELDS - 2 + par, :, pl.ds(b_lo, B_PER_GROUP)],
                osems[par],
            ).wait()
        return ()

    lax.fori_loop(0, NUM_GROUPS, group_body, ())


def kernel(indices, weight):
    idx = indices.astype(jnp.int32)
    idx = (idx % V_QUARTER) * 4 + idx // V_QUARTER
    idx = idx.reshape(NUM_WORKERS, NUM_CHUNKS, CHUNK)
    table_lin = _repack(jnp.transpose(weight)).reshape(NODES_LIN, EMBED_DIM)
    out_t = _gather_kernel(idx, table_lin)
    return jnp.transpose(out_t, (2, 0, 1))


# speedup vs baseline: 1.3580x; 1.3580x over previous
import functools

import jax
import jax.numpy as jnp
from jax import lax
from jax.experimental import pallas as pl
from jax.experimental.pallas import tpu as pltpu
from jax.experimental.pallas import tpu_sc as plsc

BATCH = 16384
FIELDS = 26
EMBED_DIM = 32
NUM_NODES = 1000000

NUM_CORES = 2
NUM_SUBCORES = 16
NUM_WORKERS = NUM_CORES * NUM_SUBCORES

TOTAL = BATCH * FIELDS
PER_WORKER = TOTAL // NUM_WORKERS
CHUNK = 128
GROUP = 13
ROWS_PER_GROUP = CHUNK * GROUP
NUM_GROUPS = PER_WORKER // ROWS_PER_GROUP
NUM_CHUNKS = PER_WORKER // CHUNK

RCOLS = 4096
RGRID = 62
V_QUARTER = RCOLS * RGRID
NODES_LIN = 4 * V_QUARTER

UB = 4096
UQ = UB // 4

assert PER_WORKER * NUM_WORKERS == TOTAL
assert ROWS_PER_GROUP * NUM_GROUPS == PER_WORKER


def _repack_block(w0, w1, w2, w3, out_ref):
    out_ref[:, 0:32] = jnp.transpose(w0[...])
    out_ref[:, 32:64] = jnp.transpose(w1[...])
    out_ref[:, 64:96] = jnp.transpose(w2[...])
    out_ref[:, 96:128] = jnp.transpose(w3[...])


def _repack(wt):
    max_blk = NUM_NODES // RCOLS
    specs = [
        pl.BlockSpec(
            (EMBED_DIM, RCOLS),
            functools.partial(
                lambda q, i: (0, jnp.minimum(q * RGRID + i, max_blk)), q
            ),
        )
        for q in range(4)
    ]
    return pl.pallas_call(
        _repack_block,
        grid=(RGRID,),
        in_specs=specs,
        out_specs=pl.BlockSpec((RCOLS, 128), lambda i: (i, 0)),
        out_shape=jax.ShapeDtypeStruct((V_QUARTER, 128), jnp.float32),
    )(wt, wt, wt, wt)


def _unpack_block(rows_ref, out_ref):
    for q in range(4):
        out_ref[0, :, q * UQ:(q + 1) * UQ] = jnp.transpose(
            rows_ref[:, q * 32:(q + 1) * 32]
        )


def _unpack(rows128):
    return pl.pallas_call(
        _unpack_block,
        grid=(FIELDS, BATCH // UB),
        in_specs=[
            pl.BlockSpec(
                (UB * EMBED_DIM // 128, 128),
                lambda f, i: (f * (BATCH // UB) + i, 0),
            )
        ],
        out_specs=pl.BlockSpec((1, EMBED_DIM, UB), lambda f, i: (f, 0, i)),
        out_shape=jax.ShapeDtypeStruct(
            (FIELDS, EMBED_DIM, BATCH), jnp.float32
        ),
    )(rows128)


@functools.partial(
    pl.kernel,
    mesh=plsc.VectorSubcoreMesh(core_axis_name="c", subcore_axis_name="s"),
    out_type=jax.ShapeDtypeStruct((TOTAL, EMBED_DIM), jnp.float32),
    scratch_types=[
        pltpu.VMEM((NUM_CHUNKS, CHUNK), jnp.int32),
        pltpu.VMEM((ROWS_PER_GROUP, EMBED_DIM), jnp.float32),
        pltpu.SemaphoreType.DMA,
        pltpu.SemaphoreType.DMA,
    ],
    compiler_params=pltpu.CompilerParams(use_tc_tiling_on_sc=False),
)
def _gather_kernel(idx_hbm, table_hbm, out_hbm, idx_v, rows_v, gsem, osem):
    wid = lax.axis_index("s") * NUM_CORES + lax.axis_index("c")
    base = wid * PER_WORKER
    pltpu.sync_copy(idx_hbm.at[wid], idx_v)

    def group_body(g, _):
        copies = []
        for j in range(GROUP):
            c = pltpu.async_copy(
                table_hbm.at[idx_v.at[g * GROUP + j]],
                rows_v.at[pl.ds(j * CHUNK, CHUNK)],
                gsem,
            )
            copies.append(c)
        for c in copies:
            c.wait()
        pltpu.async_copy(
            rows_v,
            out_hbm.at[pl.ds(base + g * ROWS_PER_GROUP, ROWS_PER_GROUP)],
            osem,
        ).wait()
        return ()

    lax.fori_loop(0, NUM_GROUPS, group_body, ())


def kernel(indices, weight):
    idx = indices.astype(jnp.int32)
    idx = (idx % V_QUARTER) * 4 + idx // V_QUARTER
    idx = jnp.transpose(idx)
    idx = idx.reshape(FIELDS, BATCH // UB, 4, UQ)
    idx = jnp.transpose(idx, (0, 1, 3, 2))
    idx = idx.reshape(NUM_WORKERS, NUM_CHUNKS, CHUNK)
    table_lin = _repack(jnp.transpose(weight)).reshape(NODES_LIN, EMBED_DIM)
    rows = _gather_kernel(idx, table_lin)
    out_t = _unpack(rows.reshape(TOTAL * EMBED_DIM // 128, 128))
    return jnp.transpose(out_t, (2, 0, 1))


# speedup vs baseline: 2.1642x; 1.5937x over previous
import functools

import jax
import jax.numpy as jnp
from jax import lax
from jax.experimental import pallas as pl
from jax.experimental.pallas import tpu as pltpu
from jax.experimental.pallas import tpu_sc as plsc

BATCH = 16384
FIELDS = 26
EMBED_DIM = 32
NUM_NODES = 1000000

NUM_CORES = 2
NUM_SUBCORES = 16
NUM_WORKERS = NUM_CORES * NUM_SUBCORES

TOTAL = BATCH * FIELDS
PER_WORKER = TOTAL // NUM_WORKERS
CHUNK = 128
GROUP = 13
ROWS_PER_GROUP = CHUNK * GROUP
NUM_GROUPS = PER_WORKER // ROWS_PER_GROUP
NUM_CHUNKS = PER_WORKER // CHUNK

RCOLS = 4096
RGRID = 62
V_QUARTER = RCOLS * RGRID
NODES_LIN = 4 * V_QUARTER

UB = 4096
UQ = UB // 4

assert PER_WORKER * NUM_WORKERS == TOTAL
assert ROWS_PER_GROUP * NUM_GROUPS == PER_WORKER


def _repack_block(w0, w1, w2, w3, out_ref):
    stacked = jnp.concatenate([w0[...], w1[...], w2[...], w3[...]], axis=0)
    out_ref[...] = jnp.transpose(stacked)


def _repack(wt):
    max_blk = NUM_NODES // RCOLS
    specs = [
        pl.BlockSpec(
            (EMBED_DIM, RCOLS),
            functools.partial(
                lambda q, i: (0, jnp.minimum(q * RGRID + i, max_blk)), q
            ),
        )
        for q in range(4)
    ]
    return pl.pallas_call(
        _repack_block,
        grid=(RGRID,),
        in_specs=specs,
        out_specs=pl.BlockSpec((RCOLS, 128), lambda i: (i, 0)),
        out_shape=jax.ShapeDtypeStruct((V_QUARTER, 128), jnp.float32),
    )(wt, wt, wt, wt)


def _unpack_block(rows_ref, out_ref):
    t = jnp.transpose(rows_ref[...])
    for q in range(4):
        out_ref[0, :, q * UQ:(q + 1) * UQ] = t[q * 32:(q + 1) * 32, :]


def _unpack(rows128):
    return pl.pallas_call(
        _unpack_block,
        grid=(FIELDS, BATCH // UB),
        in_specs=[
            pl.BlockSpec(
                (UB * EMBED_DIM // 128, 128),
                lambda f, i: (f * (BATCH // UB) + i, 0),
            )
        ],
        out_specs=pl.BlockSpec((1, EMBED_DIM, UB), lambda f, i: (f, 0, i)),
        out_shape=jax.ShapeDtypeStruct(
            (FIELDS, EMBED_DIM, BATCH), jnp.float32
        ),
    )(rows128)


@functools.partial(
    pl.kernel,
    mesh=plsc.VectorSubcoreMesh(core_axis_name="c", subcore_axis_name="s"),
    out_type=jax.ShapeDtypeStruct((TOTAL, EMBED_DIM), jnp.float32),
    scratch_types=[
        pltpu.VMEM((NUM_CHUNKS, CHUNK), jnp.int32),
        pltpu.VMEM((ROWS_PER_GROUP, EMBED_DIM), jnp.float32),
        pltpu.SemaphoreType.DMA,
        pltpu.SemaphoreType.DMA,
    ],
    compiler_params=pltpu.CompilerParams(use_tc_tiling_on_sc=False),
)
def _gather_kernel(idx_hbm, table_hbm, out_hbm, idx_v, rows_v, gsem, osem):
    wid = lax.axis_index("s") * NUM_CORES + lax.axis_index("c")
    base = wid * PER_WORKER
    pltpu.sync_copy(idx_hbm.at[wid], idx_v)

    def group_body(g, _):
        copies = []
        for j in range(GROUP):
            c = pltpu.async_copy(
                table_hbm.at[idx_v.at[g * GROUP + j]],
                rows_v.at[pl.ds(j * CHUNK, CHUNK)],
                gsem,
            )
            copies.append(c)
        for c in copies:
            c.wait()
        pltpu.async_copy(
            rows_v,
            out_hbm.at[pl.ds(base + g * ROWS_PER_GROUP, ROWS_PER_GROUP)],
            osem,
        ).wait()
        return ()

    lax.fori_loop(0, NUM_GROUPS, group_body, ())


def kernel(indices, weight):
    idx = indices.astype(jnp.int32)
    idx = (idx % V_QUARTER) * 4 + idx // V_QUARTER
    idx = jnp.transpose(idx)
    idx = idx.reshape(FIELDS, BATCH // UB, 4, UQ)
    idx = jnp.transpose(idx, (0, 1, 3, 2))
    idx = idx.reshape(NUM_WORKERS, NUM_CHUNKS, CHUNK)
    table_lin = _repack(jnp.transpose(weight)).reshape(NODES_LIN, EMBED_DIM)
    rows = _gather_kernel(idx, table_lin)
    out_t = _unpack(rows.reshape(TOTAL * EMBED_DIM // 128, 128))
    return jnp.transpose(out_t, (2, 0, 1))


# speedup vs baseline: 2.8782x; 1.3299x over previous
import functools

import jax
import jax.numpy as jnp
import numpy as np
from jax import lax
from jax.experimental import pallas as pl
from jax.experimental.pallas import tpu as pltpu
from jax.experimental.pallas import tpu_sc as plsc

BATCH = 16384
FIELDS = 26
EMBED_DIM = 32
NUM_NODES = 1000000

NUM_CORES = 2
NUM_SUBCORES = 16
NUM_WORKERS = NUM_CORES * NUM_SUBCORES

TOTAL = BATCH * FIELDS
PER_WORKER = TOTAL // NUM_WORKERS
CHUNK = 128
GROUP = 13
ROWS_PER_GROUP = CHUNK * GROUP
NUM_GROUPS = PER_WORKER // ROWS_PER_GROUP
NUM_CHUNKS = PER_WORKER // CHUNK

RCOLS = 8192
RGRID = 31
V_QUARTER = RCOLS * RGRID
NODES_LIN = 4 * V_QUARTER

UB = 8192
UQ = UB // 4

_p = np.arange(TOTAL, dtype=np.int64)
_f = _p // BATCH
_rest = _p % BATCH
_blk = _rest // UB
_r2 = _rest % UB
_b = _blk * UB + (_r2 % 4) * UQ + _r2 // 4
_IDX_PERM = jnp.asarray((_b * FIELDS + _f).astype(np.int32))

assert PER_WORKER * NUM_WORKERS == TOTAL
assert ROWS_PER_GROUP * NUM_GROUPS == PER_WORKER


def _repack_block(w0, w1, w2, w3, out_ref):
    stacked = jnp.concatenate([w0[...], w1[...], w2[...], w3[...]], axis=0)
    out_ref[...] = jnp.transpose(stacked)


def _repack(wt):
    max_blk = NUM_NODES // RCOLS
    specs = [
        pl.BlockSpec(
            (EMBED_DIM, RCOLS),
            functools.partial(
                lambda q, i: (0, jnp.minimum(q * RGRID + i, max_blk)), q
            ),
        )
        for q in range(4)
    ]
    return pl.pallas_call(
        _repack_block,
        grid=(RGRID,),
        in_specs=specs,
        out_specs=pl.BlockSpec((RCOLS, 128), lambda i: (i, 0)),
        out_shape=jax.ShapeDtypeStruct((V_QUARTER, 128), jnp.float32),
    )(wt, wt, wt, wt)


def _unpack_block(rows_ref, out_ref):
    t = jnp.transpose(rows_ref[...])
    for q in range(4):
        out_ref[0, :, q * UQ:(q + 1) * UQ] = t[q * 32:(q + 1) * 32, :]


def _unpack(rows128):
    return pl.pallas_call(
        _unpack_block,
        grid=(FIELDS, BATCH // UB),
        in_specs=[
            pl.BlockSpec(
                (UB * EMBED_DIM // 128, 128),
                lambda f, i: (f * (BATCH // UB) + i, 0),
            )
        ],
        out_specs=pl.BlockSpec((1, EMBED_DIM, UB), lambda f, i: (f, 0, i)),
        out_shape=jax.ShapeDtypeStruct(
            (FIELDS, EMBED_DIM, BATCH), jnp.float32
        ),
    )(rows128)


@functools.partial(
    pl.kernel,
    mesh=plsc.VectorSubcoreMesh(core_axis_name="c", subcore_axis_name="s"),
    out_type=jax.ShapeDtypeStruct((TOTAL, EMBED_DIM), jnp.float32),
    scratch_types=[
        pltpu.VMEM((NUM_CHUNKS, CHUNK), jnp.int32),
        pltpu.VMEM((ROWS_PER_GROUP, EMBED_DIM), jnp.float32),
        pltpu.SemaphoreType.DMA,
        pltpu.SemaphoreType.DMA,
    ],
    compiler_params=pltpu.CompilerParams(use_tc_tiling_on_sc=False),
)
def _gather_kernel(idx_hbm, table_hbm, out_hbm, idx_v, rows_v, gsem, osem):
    wid = lax.axis_index("s") * NUM_CORES + lax.axis_index("c")
    base = wid * PER_WORKER
    pltpu.sync_copy(idx_hbm.at[wid], idx_v)

    def group_body(g, _):
        copies = []
        for j in range(GROUP):
            c = pltpu.async_copy(
                table_hbm.at[idx_v.at[g * GROUP + j]],
                rows_v.at[pl.ds(j * CHUNK, CHUNK)],
                gsem,
            )
            copies.append(c)
        for c in copies:
            c.wait()
        pltpu.async_copy(
            rows_v,
            out_hbm.at[pl.ds(base + g * ROWS_PER_GROUP, ROWS_PER_GROUP)],
            osem,
        ).wait()
        return ()

    lax.fori_loop(0, NUM_GROUPS, group_body, ())


def kernel(indices, weight):
    idx = indices.reshape(TOTAL).astype(jnp.int32)[_IDX_PERM]
    idx = (idx % V_QUARTER) * 4 + idx // V_QUARTER
    idx = idx.reshape(NUM_WORKERS, NUM_CHUNKS, CHUNK)
    table_lin = _repack(jnp.transpose(weight)).reshape(NODES_LIN, EMBED_DIM)
    rows = _gather_kernel(idx, table_lin)
    out_t = _unpack(rows.reshape(TOTAL * EMBED_DIM // 128, 128))
    return jnp.transpose(out_t, (2, 0, 1))


# speedup vs baseline: 3.0974x; 1.0762x over previous
import functools

import jax
import jax.numpy as jnp
import numpy as np
from jax import lax
from jax.experimental import pallas as pl
from jax.experimental.pallas import tpu as pltpu
from jax.experimental.pallas import tpu_sc as plsc

BATCH = 16384
FIELDS = 26
EMBED_DIM = 32
NUM_NODES = 1000000

NUM_CORES = 2
NUM_SUBCORES = 16
NUM_WORKERS = NUM_CORES * NUM_SUBCORES

TOTAL = BATCH * FIELDS
PER_WORKER = TOTAL // NUM_WORKERS
CHUNK = 128
GROUP = 13
ROWS_PER_GROUP = CHUNK * GROUP
NUM_GROUPS = PER_WORKER // ROWS_PER_GROUP
NUM_CHUNKS = PER_WORKER // CHUNK

RCOLS = 12800
RGRID = 20
V_QUARTER = RCOLS * RGRID
NODES_LIN = 4 * V_QUARTER

UB = 16384
UQ = UB // 4

_p = np.arange(TOTAL, dtype=np.int64)
_f = _p // BATCH
_rest = _p % BATCH
_blk = _rest // UB
_r2 = _rest % UB
_b = _blk * UB + (_r2 % 4) * UQ + _r2 // 4
_IDX_PERM = jnp.asarray((_b * FIELDS + _f).astype(np.int32))

assert PER_WORKER * NUM_WORKERS == TOTAL
assert ROWS_PER_GROUP * NUM_GROUPS == PER_WORKER


def _repack_block(w0, w1, w2, w3, out_ref):
    stacked = jnp.concatenate([w0[...], w1[...], w2[...], w3[...]], axis=0)
    out_ref[...] = jnp.transpose(stacked)


def _repack(wt):
    max_blk = NUM_NODES // RCOLS
    specs = [
        pl.BlockSpec(
            (EMBED_DIM, RCOLS),
            functools.partial(
                lambda q, i: (0, jnp.minimum(q * RGRID + i, max_blk)), q
            ),
        )
        for q in range(4)
    ]
    return pl.pallas_call(
        _repack_block,
        grid=(RGRID,),
        in_specs=specs,
        out_specs=pl.BlockSpec((RCOLS, 128), lambda i: (i, 0)),
        out_shape=jax.ShapeDtypeStruct((V_QUARTER, 128), jnp.float32),
    )(wt, wt, wt, wt)


def _unpack_block(rows_ref, out_ref):
    t = jnp.transpose(rows_ref[...])
    for q in range(4):
        out_ref[0, :, q * UQ:(q + 1) * UQ] = t[q * 32:(q + 1) * 32, :]


def _unpack(rows128):
    return pl.pallas_call(
        _unpack_block,
        grid=(FIELDS, BATCH // UB),
        in_specs=[
            pl.BlockSpec(
                (UB * EMBED_DIM // 128, 128),
                lambda f, i: (f * (BATCH // UB) + i, 0),
            )
        ],
        out_specs=pl.BlockSpec((1, EMBED_DIM, UB), lambda f, i: (f, 0, i)),
        out_shape=jax.ShapeDtypeStruct(
            (FIELDS, EMBED_DIM, BATCH), jnp.float32
        ),
    )(rows128)


@functools.partial(
    pl.kernel,
    mesh=plsc.VectorSubcoreMesh(core_axis_name="c", subcore_axis_name="s"),
    out_type=jax.ShapeDtypeStruct((TOTAL, EMBED_DIM), jnp.float32),
    scratch_types=[
        pltpu.VMEM((NUM_CHUNKS, CHUNK), jnp.int32),
        pltpu.VMEM((ROWS_PER_GROUP, EMBED_DIM), jnp.float32),
        pltpu.SemaphoreType.DMA,
        pltpu.SemaphoreType.DMA,
    ],
    compiler_params=pltpu.CompilerParams(use_tc_tiling_on_sc=False),
)
def _gather_kernel(idx_hbm, table_hbm, out_hbm, idx_v, rows_v, gsem, osem):
    wid = lax.axis_index("s") * NUM_CORES + lax.axis_index("c")
    base = wid * PER_WORKER
    pltpu.sync_copy(idx_hbm.at[wid], idx_v)

    def group_body(g, _):
        copies = []
        for j in range(GROUP):
            c = pltpu.async_copy(
                table_hbm.at[idx_v.at[g * GROUP + j]],
                rows_v.at[pl.ds(j * CHUNK, CHUNK)],
                gsem,
            )
            copies.append(c)
        for c in copies:
            c.wait()
        pltpu.async_copy(
            rows_v,
            out_hbm.at[pl.ds(base + g * ROWS_PER_GROUP, ROWS_PER_GROUP)],
            osem,
        ).wait()
        return ()

    lax.fori_loop(0, NUM_GROUPS, group_body, ())


def kernel(indices, weight):
    idx = indices.reshape(TOTAL).astype(jnp.int32)[_IDX_PERM]
    idx = (idx % V_QUARTER) * 4 + idx // V_QUARTER
    idx = idx.reshape(NUM_WORKERS, NUM_CHUNKS, CHUNK)
    table_lin = _repack(jnp.transpose(weight)).reshape(NODES_LIN, EMBED_DIM)
    rows = _gather_kernel(idx, table_lin)
    out_t = _unpack(rows.reshape(TOTAL * EMBED_DIM // 128, 128))
    return jnp.transpose(out_t, (2, 0, 1))
